# baseline (speedup 1.0000x reference)
import functools

import jax
import jax.numpy as jnp
from jax import lax
from jax.experimental import pallas as pl
from jax.experimental.pallas import tpu as pltpu

B, C, H, W = 8, 384, 224, 224
HID, E, K = 96, 64, 2
S = H * W
CPB = 32
NCH = C // CPB
NSTEPS = B * NCH
NBUF = 6
LA = 4


def _body(x_hbm, w1_ref, b1_ref, w2_ref, b2_ref,
          idx_ref, val_ref, probs_ref, bufs, part_ref, sems):
    s = pl.program_id(0)

    def issue(j):
        jb = j // NCH
        jc = (j % NCH) * CPB
        k = j % NBUF
        pltpu.make_async_copy(
            x_hbm.at[jb, pl.ds(jc, CPB)], bufs.at[k], sems.at[k]).start()

    @pl.when(s == 0)
    def _prologue():
        for t in range(LA):
            issue(t)

    @pl.when(s + LA < NSTEPS)
    def _ahead():
        issue(s + LA)

    k = s % NBUF
    b = s // NCH
    ci = s % NCH
    pltpu.make_async_copy(
        x_hbm.at[b, pl.ds(ci * CPB, CPB)], bufs.at[k], sems.at[k]).wait()
    psum = jnp.sum(bufs[k], axis=(1, 2))[None, :]
    part_ref[pl.ds(b, 1), ci, :] = psum

    @pl.when(s == NSTEPS - 1)
    def _router():
        hid = jnp.zeros((B, HID), jnp.float32)
        for cj in range(NCH):
            hcj = part_ref[:, cj, :] * (1.0 / S)
            hid += jnp.dot(hcj, w1_ref[cj], preferred_element_type=jnp.float32)
        hid = jnp.maximum(hid + b1_ref[...], 0.0)
        logits = jnp.dot(hid, w2_ref[...], preferred_element_type=jnp.float32)
        logits = logits + b2_ref[...]
        m = jnp.max(logits, axis=1, keepdims=True)
        e = jnp.exp(logits - m)
        p = e / jnp.sum(e, axis=1, keepdims=True)
        probs_ref[...] = p
        iota = lax.broadcasted_iota(jnp.int32, p.shape, 1)
        m1 = jnp.max(p, axis=1, keepdims=True)
        i1 = jnp.min(jnp.where(p == m1, iota, E), axis=1, keepdims=True)
        p2 = jnp.where(iota == i1, -jnp.inf, p)
        m2 = jnp.max(p2, axis=1, keepdims=True)
        i2 = jnp.min(jnp.where(p2 == m2, iota, E), axis=1, keepdims=True)
        val_ref[...] = jnp.concatenate([m1, m2], axis=1)
        idx_ref[...] = jnp.concatenate([i1, i2], axis=1)


@jax.jit
def kernel(x, W1, b1, W2, b2):
    w1t = W1.T.reshape(NCH, CPB, HID)
    w2t = W2.T
    b1r = b1.reshape(1, HID)
    b2r = b2.reshape(1, E)

    out = pl.pallas_call(
        _body,
        grid=(NSTEPS,),
        in_specs=[pl.BlockSpec(memory_space=pl.ANY),
                  pl.BlockSpec((NCH, CPB, HID), lambda s: (0, 0, 0)),
                  pl.BlockSpec((1, HID), lambda s: (0, 0)),
                  pl.BlockSpec((HID, E), lambda s: (0, 0)),
                  pl.BlockSpec((1, E), lambda s: (0, 0))],
        out_specs=[
            pl.BlockSpec((B, K), lambda s: (0, 0)),
            pl.BlockSpec((B, K), lambda s: (0, 0)),
            pl.BlockSpec((B, E), lambda s: (0, 0)),
        ],
        out_shape=[
            jax.ShapeDtypeStruct((B, K), jnp.int32),
            jax.ShapeDtypeStruct((B, K), jnp.float32),
            jax.ShapeDtypeStruct((B, E), jnp.float32),
        ],
        scratch_shapes=[
            pltpu.VMEM((NBUF, CPB, H, W), jnp.float32),
            pltpu.VMEM((B, NCH, CPB), jnp.float32),
            pltpu.SemaphoreType.DMA((NBUF,)),
        ],
        compiler_params=pltpu.CompilerParams(
            dimension_semantics=("arbitrary",),
        ),
    )(x, w1t, b1r, w2t, b2r)
    topk_idx, topk_val, probs = out
    return (topk_idx, topk_val, probs)


# speedup vs baseline: 4.5586x; 4.5586x over previous
import functools

import jax
import jax.numpy as jnp
from jax import lax
from jax.experimental import pallas as pl
from jax.experimental.pallas import tpu as pltpu

B, C, H, W = 8, 384, 224, 224
HID, E, K = 96, 64, 2
S = H * W
HB = 28
NH = H // HB


def _body(x_ref, w1_ref, b1_ref, w2_ref, b2_ref,
          idx_ref, val_ref, probs_ref, part_ref):
    b = pl.program_id(0)
    hb = pl.program_id(1)
    psum = jnp.sum(x_ref[...], axis=(1, 2))

    @pl.when(hb == 0)
    def _init():
        part_ref[pl.ds(b, 1), :] = psum

    @pl.when(hb != 0)
    def _acc():
        part_ref[pl.ds(b, 1), :] += psum

    @pl.when((b == B - 1) & (hb == NH - 1))
    def _router():
        h = part_ref[...] * (1.0 / S)
        hid = lax.dot_general(h, w1_ref[...], (((1,), (1,)), ((), ())),
                              preferred_element_type=jnp.float32)
        hid = jnp.maximum(hid + b1_ref[...], 0.0)
        logits = lax.dot_general(hid, w2_ref[...], (((1,), (1,)), ((), ())),
                                 preferred_element_type=jnp.float32)
        logits = logits + b2_ref[...]
        m = jnp.max(logits, axis=1, keepdims=True)
        e = jnp.exp(logits - m)
        p = e / jnp.sum(e, axis=1, keepdims=True)
        probs_ref[...] = p
        iota = lax.broadcasted_iota(jnp.int32, p.shape, 1)
        m1 = jnp.max(p, axis=1, keepdims=True)
        i1 = jnp.min(jnp.where(p == m1, iota, E), axis=1, keepdims=True)
        p2 = jnp.where(iota == i1, -jnp.inf, p)
        m2 = jnp.max(p2, axis=1, keepdims=True)
        i2 = jnp.min(jnp.where(p2 == m2, iota, E), axis=1, keepdims=True)
        val_ref[...] = jnp.concatenate([m1, m2], axis=1)
        idx_ref[...] = jnp.concatenate([i1, i2], axis=1)


@jax.jit
def kernel(x, W1, b1, W2, b2):
    xt = jnp.transpose(x, (0, 2, 3, 1))
    b1r = b1.reshape(1, HID)
    b2r = b2.reshape(1, E)

    out = pl.pallas_call(
        _body,
        grid=(B, NH),
        in_specs=[pl.BlockSpec((1, HB, W, C), lambda b, hb: (b, hb, 0, 0)),
                  pl.BlockSpec((HID, C), lambda b, hb: (0, 0)),
                  pl.BlockSpec((1, HID), lambda b, hb: (0, 0)),
                  pl.BlockSpec((E, HID), lambda b, hb: (0, 0)),
                  pl.BlockSpec((1, E), lambda b, hb: (0, 0))],
        out_specs=[
            pl.BlockSpec((B, K), lambda b, hb: (0, 0)),
            pl.BlockSpec((B, K), lambda b, hb: (0, 0)),
            pl.BlockSpec((B, E), lambda b, hb: (0, 0)),
        ],
        out_shape=[
            jax.ShapeDtypeStruct((B, K), jnp.int32),
            jax.ShapeDtypeStruct((B, K), jnp.float32),
            jax.ShapeDtypeStruct((B, E), jnp.float32),
        ],
        scratch_shapes=[pltpu.VMEM((B, C), jnp.float32)],
        compiler_params=pltpu.CompilerParams(
            dimension_semantics=("arbitrary", "arbitrary"),
        ),
    )(xt, W1, b1r, W2, b2r)
    topk_idx, topk_val, probs = out
    return (topk_idx, topk_val, probs)
